# baseline (speedup 1.0000x reference)
import functools
import jax
import jax.numpy as jnp
from jax import lax
from jax.experimental import pallas as pl
from jax.experimental.pallas import tpu as pltpu
from jax.experimental.pallas import tpu_sc as plsc

_IOU_THR = 0.5
_CONF_THR = 0.05
_MAX_DET = 100
_N_PAD = 1024
_T_PAD = 128

f32 = jnp.float32
i32 = jnp.int32


def _softmax_body(conf_ref, box_ref, s_out, area_out):
    B, C, NP = conf_ref.shape
    N = 1000
    z = conf_ref[...]
    zmax = jnp.max(z, axis=1, keepdims=True)
    e = jnp.exp(z - zmax)
    se = jnp.sum(e, axis=1, keepdims=True)
    s = e / se
    n_io3 = lax.broadcasted_iota(i32, (B, C, NP), 2)
    s_out[...] = jnp.where(n_io3 < N, s, f32(-jnp.inf))
    y1 = box_ref[:, 0, :]
    x1 = box_ref[:, 1, :]
    y2 = box_ref[:, 2, :]
    x2 = box_ref[:, 3, :]
    area_out[...] = jnp.maximum(y2 - y1, 0.0) * jnp.maximum(x2 - x1, 0.0)


def _sc_nms_body(scores_h, y1_h, x1_h, y2_h, x2_h, area_h,
                 cs_h, ci_h,
                 s_v, y1_v, x1_v, y2_v, x2_v, area_v, csv, civ):
    C = 80
    TASKS_PER = C // 4
    NV = _N_PAD // 16
    NEG = f32(-jnp.inf)
    lane = lax.iota(i32, 16)

    wid = lax.axis_index("s") * 2 + lax.axis_index("c")
    b = wid // 4
    c0 = (wid % 4) * TASKS_PER

    pltpu.sync_copy(y1_h.at[b], y1_v)
    pltpu.sync_copy(x1_h.at[b], x1_v)
    pltpu.sync_copy(y2_h.at[b], y2_v)
    pltpu.sync_copy(x2_h.at[b], x2_v)
    pltpu.sync_copy(area_h.at[b], area_v)

    def task_body(ti, _):
        t = b * C + c0 + ti
        pltpu.sync_copy(scores_h.at[t], s_v)
        for j in range(_T_PAD // 16):
            csv[pl.ds(j * 16, 16)] = jnp.full((16,), -1.0, f32)
            civ[pl.ds(j * 16, 16)] = jnp.zeros((16,), i32)

        def amx(i, carry):
            m, nb = carry
            v = s_v[pl.ds(i * 16, 16)]
            nidx = jnp.full((16,), i * 16, i32) + lane
            upd = v > m
            return jnp.where(upd, v, m), jnp.where(upd, nidx, nb)

        m, nb = lax.fori_loop(
            0, NV, amx,
            (jnp.full((16,), NEG, f32), jnp.zeros((16,), i32)))
        M = jnp.max(m)
        nstar = jnp.min(jnp.where(m == M, nb, 1 << 20))

        def g_cond(carry):
            tk, Mc, _ = carry
            return (Mc > _CONF_THR) & (tk < _MAX_DET)

        def g_body(carry):
            tk, Mc, n = carry
            tkv = jnp.full((16,), tk, i32)
            lane0 = lane == 0
            plsc.store_scatter(csv, [tkv], jnp.full((16,), Mc, f32),
                               mask=lane0)
            plsc.store_scatter(civ, [tkv], jnp.full((16,), n, i32),
                               mask=lane0)
            nsplat = jnp.full((16,), n, i32)
            sy1 = plsc.load_gather(y1_v, [nsplat])
            sx1 = plsc.load_gather(x1_v, [nsplat])
            sy2 = plsc.load_gather(y2_v, [nsplat])
            sx2 = plsc.load_gather(x2_v, [nsplat])
            asel = (jnp.maximum(sy2 - sy1, 0.0) *
                    jnp.maximum(sx2 - sx1, 0.0))

            def sup(i, carry2):
                m2, nb2 = carry2
                sl = pl.ds(i * 16, 16)
                sv = s_v[sl]
                iy1 = jnp.maximum(sy1, y1_v[sl])
                ix1 = jnp.maximum(sx1, x1_v[sl])
                iy2 = jnp.minimum(sy2, y2_v[sl])
                ix2 = jnp.minimum(sx2, x2_v[sl])
                inter = (jnp.maximum(iy2 - iy1, 0.0) *
                         jnp.maximum(ix2 - ix1, 0.0))
                union = asel + area_v[sl] - inter
                iou = inter / (union + 1e-8)
                nidx = jnp.full((16,), i * 16, i32) + lane
                kill = (iou > _IOU_THR) | (nidx == nsplat)
                sv2 = jnp.where(kill, NEG, sv)
                s_v[sl] = sv2
                upd = sv2 > m2
                return jnp.where(upd, sv2, m2), jnp.where(upd, nidx, nb2)

            m2, nb2 = lax.fori_loop(
                0, NV, sup,
                (jnp.full((16,), NEG, f32), jnp.zeros((16,), i32)))
            M2 = jnp.max(m2)
            n2 = jnp.min(jnp.where(m2 == M2, nb2, 1 << 20))
            return tk + 1, M2, n2

        lax.while_loop(g_cond, g_body, (jnp.array(0, i32), M, nstar))
        pltpu.sync_copy(csv, cs_h.at[t])
        pltpu.sync_copy(civ, ci_h.at[t])
        return 0

    lax.fori_loop(0, TASKS_PER, task_body, 0)


def _run_sc_stage(scores2d, y1, x1, y2, x2, area):
    BC, NP = scores2d.shape
    B = y1.shape[0]
    sc_nms = functools.partial(
        pl.kernel,
        mesh=plsc.VectorSubcoreMesh(core_axis_name="c", subcore_axis_name="s"),
        compiler_params=pltpu.CompilerParams(needs_layout_passes=False),
        out_type=[
            jax.ShapeDtypeStruct((BC, _T_PAD), f32),
            jax.ShapeDtypeStruct((BC, _T_PAD), i32),
        ],
        scratch_types=[
            pltpu.VMEM((NP,), f32),
            pltpu.VMEM((NP,), f32),
            pltpu.VMEM((NP,), f32),
            pltpu.VMEM((NP,), f32),
            pltpu.VMEM((NP,), f32),
            pltpu.VMEM((NP,), f32),
            pltpu.VMEM((_T_PAD,), f32),
            pltpu.VMEM((_T_PAD,), i32),
        ],
    )(_sc_nms_body)
    return sc_nms(scores2d, y1, x1, y2, x2, area)


def _topk_body(cs_in, ci_in, box_ref, conf_o, cls_o, box_o, num_o, cs_ref):
    B, C, TP = cs_in.shape
    NP = box_ref.shape[2]
    T = _MAX_DET
    NEG = f32(-jnp.inf)

    conf_o[...] = jnp.zeros(conf_o.shape, f32)
    cls_o[...] = jnp.zeros(cls_o.shape, f32)
    box_o[...] = jnp.zeros(box_o.shape, f32)
    cs_ref[...] = cs_in[...]

    t_io3 = lax.broadcasted_iota(i32, (B, C, TP), 2)
    c_io3 = lax.broadcasted_iota(i32, (B, C, TP), 1)
    c_io2 = lax.broadcasted_iota(i32, (B, C), 1)
    n_io2 = lax.broadcasted_iota(i32, (B, NP), 1)

    def topk_cond(c):
        k, go = c
        return go & (k < T)

    def topk_step(c):
        k, _ = c
        cs = cs_ref[...]
        m_t = jnp.max(cs, axis=2)
        tstar = jnp.min(jnp.where(cs == m_t[:, :, None], t_io3, TP),
                        axis=2)
        m_b = jnp.max(m_t, axis=1)
        cstar = jnp.min(jnp.where(m_t == m_b[:, None], c_io2, C), axis=1)
        tsel = jnp.min(jnp.where(c_io2 == cstar[:, None], tstar, 10000),
                       axis=1)
        oh3 = ((t_io3 == tsel[:, None, None]) &
               (c_io3 == cstar[:, None, None]))
        bidx = jnp.sum(jnp.sum(jnp.where(oh3, ci_in[...], 0), axis=2),
                       axis=1)
        cs_ref[...] = jnp.where(oh3, NEG, cs)
        valid = m_b > 0.0
        conf_o[k] = jnp.where(valid, m_b, 0.0).reshape(1, B)
        cls_o[k] = jnp.where(valid, cstar.astype(f32), 0.0).reshape(1, B)
        ohn = (n_io2 == bidx[:, None]) & valid[:, None]
        rows = [
            jnp.sum(jnp.where(ohn, box_ref[:, j, :], 0.0),
                    axis=1).reshape(1, B)
            for j in range(4)
        ]
        box_o[k] = jnp.concatenate(rows, axis=0)
        return k + 1, jnp.any(valid)

    lax.while_loop(topk_cond, topk_step,
                   (jnp.array(0, i32), jnp.array(True)))
    num_o[...] = jnp.sum((conf_o[...] > 0.0).astype(i32), axis=0)


def kernel(box_pred, confidence_pred):
    B, N, C = confidence_pred.shape
    NP = _N_PAD
    T = _MAX_DET
    conf_t = jnp.pad(jnp.transpose(confidence_pred, (0, 2, 1)),
                     ((0, 0), (0, 0), (0, NP - N)))
    box_t = jnp.pad(jnp.transpose(box_pred, (0, 2, 1)),
                    ((0, 0), (0, 0), (0, NP - N)))

    scores, area = pl.pallas_call(
        _softmax_body,
        out_shape=[
            jax.ShapeDtypeStruct((B, C, NP), f32),
            jax.ShapeDtypeStruct((B, NP), f32),
        ],
    )(conf_t, box_t)

    cand_s, cand_i = _run_sc_stage(
        scores.reshape(B * C, NP),
        box_t[:, 0, :], box_t[:, 1, :], box_t[:, 2, :], box_t[:, 3, :],
        area)

    conf_o, cls_o, box_o, num_o = pl.pallas_call(
        _topk_body,
        out_shape=[
            jax.ShapeDtypeStruct((T, 1, B), f32),
            jax.ShapeDtypeStruct((T, 1, B), f32),
            jax.ShapeDtypeStruct((T, 4, B), f32),
            jax.ShapeDtypeStruct((1, B), jnp.int32),
        ],
        scratch_shapes=[
            pltpu.VMEM((B, C, _T_PAD), f32),
        ],
    )(cand_s.reshape(B, C, _T_PAD), cand_i.reshape(B, C, _T_PAD), box_t)

    boxes_out = jnp.transpose(box_o, (2, 0, 1))
    conf_out = conf_o[:, 0, :].T
    cls_out = cls_o[:, 0, :].T
    num = num_o[0]
    return boxes_out, conf_out, cls_out, num


# speedup vs baseline: 3.7754x; 3.7754x over previous
import functools
import jax
import jax.numpy as jnp
from jax import lax
from jax.experimental import pallas as pl
from jax.experimental.pallas import tpu as pltpu
from jax.experimental.pallas import tpu_sc as plsc

_IOU_THR = 0.5
_CONF_THR = 0.05
_MAX_DET = 100
_N_PAD = 1024
_NC_PAD = _N_PAD + 16
_T_PAD = 128

f32 = jnp.float32
i32 = jnp.int32


def _softmax_body(conf_ref, box_ref, s_out, area_out):
    B, C, NP = conf_ref.shape
    N = 1000
    z = conf_ref[...]
    zmax = jnp.max(z, axis=1, keepdims=True)
    e = jnp.exp(z - zmax)
    se = jnp.sum(e, axis=1, keepdims=True)
    s = e / se
    n_io3 = lax.broadcasted_iota(i32, (B, C, NP), 2)
    s_out[...] = jnp.where(n_io3 < N, s, f32(-jnp.inf))
    y1 = box_ref[:, 0, :]
    x1 = box_ref[:, 1, :]
    y2 = box_ref[:, 2, :]
    x2 = box_ref[:, 3, :]
    area_out[...] = jnp.maximum(y2 - y1, 0.0) * jnp.maximum(x2 - x1, 0.0)


def _sc_nms_body(scores_h, y1_h, x1_h, y2_h, x2_h, area_h,
                 cs_h, ci_h,
                 s_v, y1_v, x1_v, y2_v, x2_v, area_v,
                 sc_c, idx_c, y1c, x1c, y2c, x2c, areac, csv, civ):
    C = 80
    TASKS_PER = C // 4
    NV = _N_PAD // 16
    NEG = f32(-jnp.inf)
    lane = lax.iota(i32, 16)
    zero16f = jnp.zeros((16,), f32)
    zero16i = jnp.zeros((16,), i32)

    wid = lax.axis_index("s") * 2 + lax.axis_index("c")
    b = wid // 4
    c0 = (wid % 4) * TASKS_PER

    pltpu.sync_copy(y1_h.at[b], y1_v)
    pltpu.sync_copy(x1_h.at[b], x1_v)
    pltpu.sync_copy(y2_h.at[b], y2_v)
    pltpu.sync_copy(x2_h.at[b], x2_v)
    pltpu.sync_copy(area_h.at[b], area_v)

    def task_body(ti, _):
        t = b * C + c0 + ti
        pltpu.sync_copy(scores_h.at[t], s_v)
        for j in range(_T_PAD // 16):
            csv[pl.ds(j * 16, 16)] = jnp.full((16,), -1.0, f32)
            civ[pl.ds(j * 16, 16)] = zero16i

        for j in range(_NC_PAD // 16):
            sc_c[pl.ds(j * 16, 16)] = jnp.full((16,), NEG, f32)
            idx_c[pl.ds(j * 16, 16)] = zero16i
        def compact(i, offv):
            sv = s_v[pl.ds(i * 16, 16)]
            msk = sv > _CONF_THR
            cum = plsc.cumsum(msk.astype(i32))
            pos = offv + cum - 1
            plsc.store_scatter(sc_c, [pos], sv, mask=msk)
            nidx = jnp.full((16,), i * 16, i32) + lane
            plsc.store_scatter(idx_c, [pos], nidx, mask=msk)
            return offv + plsc.all_reduce_population_count(msk)

        offv = lax.fori_loop(0, NV, compact, zero16i)
        K = jnp.max(offv)
        nv = (K + 15) // 16
        pass

        def gatherc(j, _):
            sl = pl.ds(j * 16, 16)
            idxv = idx_c[sl]
            y1c[sl] = plsc.load_gather(y1_v, [idxv])
            x1c[sl] = plsc.load_gather(x1_v, [idxv])
            y2c[sl] = plsc.load_gather(y2_v, [idxv])
            x2c[sl] = plsc.load_gather(x2_v, [idxv])
            areac[sl] = plsc.load_gather(area_v, [idxv])
            return 0

        lax.fori_loop(0, nv, gatherc, 0)

        def amx(j, carry):
            m, pb = carry
            v = sc_c[pl.ds(j * 16, 16)]
            posv = jnp.full((16,), j * 16, i32) + lane
            upd = v > m
            return jnp.where(upd, v, m), jnp.where(upd, posv, pb)

        m, pb = lax.fori_loop(0, nv, amx,
                              (jnp.full((16,), NEG, f32), zero16i))
        M = jnp.max(m)
        pstar = jnp.min(jnp.where(m == M, pb, 1 << 20))

        def g_cond(carry):
            tk, Mc, _ = carry
            return (Mc > _CONF_THR) & (tk < _MAX_DET)

        def g_body(carry):
            tk, Mc, p = carry
            tkv = jnp.full((16,), tk, i32)
            lane0 = lane == 0
            psplat = jnp.full((16,), p, i32)
            orig = plsc.load_gather(idx_c, [psplat])
            plsc.store_scatter(csv, [tkv], jnp.full((16,), Mc, f32),
                               mask=lane0)
            plsc.store_scatter(civ, [tkv], orig, mask=lane0)
            sy1 = plsc.load_gather(y1c, [psplat])
            sx1 = plsc.load_gather(x1c, [psplat])
            sy2 = plsc.load_gather(y2c, [psplat])
            sx2 = plsc.load_gather(x2c, [psplat])
            asel = plsc.load_gather(areac, [psplat])

            def sup(j, carry2):
                m2, pb2 = carry2
                sl = pl.ds(j * 16, 16)
                sv = sc_c[sl]
                iy1 = jnp.maximum(sy1, y1c[sl])
                ix1 = jnp.maximum(sx1, x1c[sl])
                iy2 = jnp.minimum(sy2, y2c[sl])
                ix2 = jnp.minimum(sx2, x2c[sl])
                inter = (jnp.maximum(iy2 - iy1, 0.0) *
                         jnp.maximum(ix2 - ix1, 0.0))
                union = asel + areac[sl] - inter
                iou = inter / (union + 1e-8)
                posv = jnp.full((16,), j * 16, i32) + lane
                kill = (iou > _IOU_THR) | (posv == psplat)
                sv2 = jnp.where(kill, NEG, sv)
                sc_c[sl] = sv2
                upd = sv2 > m2
                return jnp.where(upd, sv2, m2), jnp.where(upd, posv, pb2)

            m2, pb2 = lax.fori_loop(0, nv, sup,
                                    (jnp.full((16,), NEG, f32), zero16i))
            M2 = jnp.max(m2)
            p2 = jnp.min(jnp.where(m2 == M2, pb2, 1 << 20))
            return tk + 1, M2, p2

        lax.while_loop(g_cond, g_body, (jnp.array(0, i32), M, pstar))
        pltpu.sync_copy(csv, cs_h.at[t])
        pltpu.sync_copy(civ, ci_h.at[t])
        return 0

    lax.fori_loop(0, TASKS_PER, task_body, 0)


def _run_sc_stage(scores2d, y1, x1, y2, x2, area):
    BC, NP = scores2d.shape
    sc_nms = functools.partial(
        pl.kernel,
        mesh=plsc.VectorSubcoreMesh(core_axis_name="c", subcore_axis_name="s"),
        compiler_params=pltpu.CompilerParams(needs_layout_passes=False),
        out_type=[
            jax.ShapeDtypeStruct((BC, _T_PAD), f32),
            jax.ShapeDtypeStruct((BC, _T_PAD), i32),
        ],
        scratch_types=[
            pltpu.VMEM((NP,), f32),
            pltpu.VMEM((NP,), f32),
            pltpu.VMEM((NP,), f32),
            pltpu.VMEM((NP,), f32),
            pltpu.VMEM((NP,), f32),
            pltpu.VMEM((NP,), f32),
            pltpu.VMEM((_NC_PAD,), f32),
            pltpu.VMEM((_NC_PAD,), i32),
            pltpu.VMEM((_NC_PAD,), f32),
            pltpu.VMEM((_NC_PAD,), f32),
            pltpu.VMEM((_NC_PAD,), f32),
            pltpu.VMEM((_NC_PAD,), f32),
            pltpu.VMEM((_NC_PAD,), f32),
            pltpu.VMEM((_T_PAD,), f32),
            pltpu.VMEM((_T_PAD,), i32),
        ],
    )(_sc_nms_body)
    return sc_nms(scores2d, y1, x1, y2, x2, area)


def _topk_body(cs_in, ci_in, box_ref, conf_o, cls_o, box_o, num_o, cs_ref):
    B, C, TP = cs_in.shape
    NP = box_ref.shape[2]
    T = _MAX_DET
    NEG = f32(-jnp.inf)

    conf_o[...] = jnp.zeros(conf_o.shape, f32)
    cls_o[...] = jnp.zeros(cls_o.shape, f32)
    box_o[...] = jnp.zeros(box_o.shape, f32)
    cs_ref[...] = cs_in[...]

    t_io3 = lax.broadcasted_iota(i32, (B, C, TP), 2)
    c_io3 = lax.broadcasted_iota(i32, (B, C, TP), 1)
    c_io2 = lax.broadcasted_iota(i32, (B, C), 1)
    n_io2 = lax.broadcasted_iota(i32, (B, NP), 1)

    def topk_cond(c):
        k, go = c
        return go & (k < T)

    def topk_step(c):
        k, _ = c
        cs = cs_ref[...]
        m_t = jnp.max(cs, axis=2)
        tstar = jnp.min(jnp.where(cs == m_t[:, :, None], t_io3, TP),
                        axis=2)
        m_b = jnp.max(m_t, axis=1)
        cstar = jnp.min(jnp.where(m_t == m_b[:, None], c_io2, C), axis=1)
        tsel = jnp.min(jnp.where(c_io2 == cstar[:, None], tstar, 10000),
                       axis=1)
        oh3 = ((t_io3 == tsel[:, None, None]) &
               (c_io3 == cstar[:, None, None]))
        bidx = jnp.sum(jnp.sum(jnp.where(oh3, ci_in[...], 0), axis=2),
                       axis=1)
        cs_ref[...] = jnp.where(oh3, NEG, cs)
        valid = m_b > 0.0
        conf_o[k] = jnp.where(valid, m_b, 0.0).reshape(1, B)
        cls_o[k] = jnp.where(valid, cstar.astype(f32), 0.0).reshape(1, B)
        ohn = (n_io2 == bidx[:, None]) & valid[:, None]
        rows = [
            jnp.sum(jnp.where(ohn, box_ref[:, j, :], 0.0),
                    axis=1).reshape(1, B)
            for j in range(4)
        ]
        box_o[k] = jnp.concatenate(rows, axis=0)
        return k + 1, jnp.any(valid)

    lax.while_loop(topk_cond, topk_step,
                   (jnp.array(0, i32), jnp.array(True)))
    num_o[...] = jnp.sum((conf_o[...] > 0.0).astype(i32), axis=0)


def kernel(box_pred, confidence_pred):
    B, N, C = confidence_pred.shape
    NP = _N_PAD
    T = _MAX_DET
    conf_t = jnp.pad(jnp.transpose(confidence_pred, (0, 2, 1)),
                     ((0, 0), (0, 0), (0, NP - N)))
    box_t = jnp.pad(jnp.transpose(box_pred, (0, 2, 1)),
                    ((0, 0), (0, 0), (0, NP - N)))

    scores, area = pl.pallas_call(
        _softmax_body,
        out_shape=[
            jax.ShapeDtypeStruct((B, C, NP), f32),
            jax.ShapeDtypeStruct((B, NP), f32),
        ],
    )(conf_t, box_t)

    cand_s, cand_i = _run_sc_stage(
        scores.reshape(B * C, NP),
        box_t[:, 0, :], box_t[:, 1, :], box_t[:, 2, :], box_t[:, 3, :],
        area)

    conf_o, cls_o, box_o, num_o = pl.pallas_call(
        _topk_body,
        out_shape=[
            jax.ShapeDtypeStruct((T, 1, B), f32),
            jax.ShapeDtypeStruct((T, 1, B), f32),
            jax.ShapeDtypeStruct((T, 4, B), f32),
            jax.ShapeDtypeStruct((1, B), jnp.int32),
        ],
        scratch_shapes=[
            pltpu.VMEM((B, C, _T_PAD), f32),
        ],
    )(cand_s.reshape(B, C, _T_PAD), cand_i.reshape(B, C, _T_PAD), box_t)

    boxes_out = jnp.transpose(box_o, (2, 0, 1))
    conf_out = conf_o[:, 0, :].T
    cls_out = cls_o[:, 0, :].T
    num = num_o[0]
    return boxes_out, conf_out, cls_out, num
